# baseline (speedup 1.0000x reference)
import functools

import jax
import jax.numpy as jnp
from jax import lax
from jax.experimental import pallas as pl
from jax.experimental.pallas import tpu as pltpu
from jax.experimental.pallas import tpu_sc as plsc

K = 100000
DIM = 128
B = 16384

_INFO = plsc.get_sparse_core_info()
_NC, _NS = _INFO.num_cores, _INFO.num_subcores
_NW = _NC * _NS

_CH = 256
_VCH = B // _CH
_NCHUNK = _VCH + (K - B + _CH - 1) // _CH
_NITER = (_NCHUNK + _NW - 1) // _NW


@functools.partial(
    pl.kernel,
    mesh=plsc.VectorSubcoreMesh(core_axis_name="c", subcore_axis_name="s"),
    out_type=jax.ShapeDtypeStruct((K, DIM), jnp.float32),
    scratch_types=[
        pltpu.VMEM((_CH, DIM), jnp.float32),
        pltpu.VMEM((_CH, DIM), jnp.float32),
        pltpu.VMEM((_CH, DIM), jnp.float32),
        pltpu.SemaphoreType.DMA,
        pltpu.SemaphoreType.DMA,
        pltpu.SemaphoreType.DMA,
        pltpu.SemaphoreType.DMA,
        pltpu.SemaphoreType.DMA,
        pltpu.SemaphoreType.DMA,
    ],
)
def _push(bank_hbm, value_hbm, out_hbm, buf0, buf1, buf2,
          si0, si1, si2, so0, so1, so2):
    wid = lax.axis_index("s") * _NC + lax.axis_index("c")
    bufs, sin, sout = (buf0, buf1, buf2), (si0, si1, si2), (so0, so1, so2)

    def chunk_row(i):
        c = jnp.minimum(wid + i * _NW, _NCHUNK - 1)
        row = jnp.minimum(c * _CH, K - _CH)
        return c, pl.multiple_of(row, 8)

    def start_in(i):
        b = i % 3
        c, row = chunk_row(i)

        @pl.when(c < _VCH)
        def _():
            pltpu.make_async_copy(
                value_hbm.at[pl.ds(row, _CH)], bufs[b], sin[b]).start()

        @pl.when(c >= _VCH)
        def _():
            pltpu.make_async_copy(
                bank_hbm.at[pl.ds(row, _CH)], bufs[b], sin[b]).start()

    out_handles = [None, None, None]
    start_in(0)
    for i in range(_NITER):
        b = i % 3
        if i + 1 < _NITER:
            bn = (i + 1) % 3
            if out_handles[bn] is not None:
                out_handles[bn].wait()
                out_handles[bn] = None
            start_in(i + 1)
        _, row = chunk_row(i)
        pltpu.make_async_copy(
            bank_hbm.at[pl.ds(row, _CH)], bufs[b], sin[b]).wait()

        oh = pltpu.make_async_copy(bufs[b], out_hbm.at[pl.ds(row, _CH)], sout[b])
        oh.start()
        out_handles[b] = oh
    for oh in out_handles:
        if oh is not None:
            oh.wait()


def kernel(bank, value):
    return _push(bank, value)


# speedup vs baseline: 2.0149x; 2.0149x over previous
import jax
import jax.numpy as jnp
from jax.experimental import pallas as pl
from jax.experimental.pallas import tpu as pltpu

K = 100000
DIM = 128
B = 16384

_CHUNKS = ((0, B), (B, 32768), (B + 32768, 32768), (B + 65536, K - B - 65536))


def _push_body(bank_ref, value_ref, out_ref, *scratch):
    n = len(_CHUNKS)
    bufs, sin, sout = scratch[:n], scratch[n:2 * n], scratch[2 * n:]
    ins, outs = [], []
    for i, (r0, nr) in enumerate(_CHUNKS):
        src = value_ref.at[pl.ds(0, nr)] if i == 0 else bank_ref.at[pl.ds(r0, nr)]
        ins.append(pltpu.make_async_copy(src, bufs[i], sin[i]))
        outs.append(pltpu.make_async_copy(
            bufs[i], out_ref.at[pl.ds(r0, nr)], sout[i]))
    for c in ins:
        c.start()
    for i in range(n):
        ins[i].wait()
        outs[i].start()
    for c in outs:
        c.wait()


@jax.jit
def kernel(bank, value):
    return pl.pallas_call(
        _push_body,
        out_shape=jax.ShapeDtypeStruct((K, DIM), jnp.float32),
        in_specs=[
            pl.BlockSpec(memory_space=pl.ANY),
            pl.BlockSpec(memory_space=pl.ANY),
        ],
        out_specs=pl.BlockSpec(memory_space=pl.ANY),
        scratch_shapes=(
            [pltpu.VMEM((nr, DIM), jnp.float32) for _, nr in _CHUNKS]
            + [pltpu.SemaphoreType.DMA] * (2 * len(_CHUNKS))
        ),
    )(bank, value)
